# baseline (speedup 1.0000x reference)
import jax
import jax.numpy as jnp
from jax import lax
from jax.experimental import pallas as pl
from jax.experimental.pallas import tpu as pltpu

B, N1, N2, DIM = 4, 256, 256, 128
HID = 128
OUT1, OUT2 = 3, 3
NOUT = OUT1 + OUT2
TQ = 16
SUB = 8
NSUB = TQ // SUB
NSTEP = N1 // TQ
RH = SUB * N2


def _pair_kernel(nq_ref, q_ref, d_ref, qmf_ref, qmc_ref, dm_ref, tri_ref,
                 tril_ref, w0q_ref, w0d_ref, b0_ref, w12a_ref, b12a_ref,
                 wbt_ref, bout_ref, out_ref, aq_ref, bd_ref, ssc_ref):
    i = pl.program_id(1)
    nq = nq_ref[0, 0, 0, 0]

    @pl.when(i == 0)
    def _():
        ssc_ref[...] = jnp.zeros((NOUT, N1, N2), jnp.float32)

        bd_ref[...] = (
            jnp.dot(d_ref[0].astype(jnp.bfloat16), w0d_ref[...],
                    preferred_element_type=jnp.float32) + b0_ref[...]
        )

        qmv = qmf_ref[0]
        cum = jnp.dot(qmv.astype(jnp.bfloat16), tri_ref[...],
                      preferred_element_type=jnp.float32)
        nqv = jnp.sum(qmv, axis=1, keepdims=True)
        lane = lax.broadcasted_iota(jnp.int32, (1, N1), 1).astype(jnp.float32)
        pos = jnp.where(qmv > 0.0, cum - 1.0, nqv + lane - cum)
        pos_i = pos.astype(jnp.int32)

        rowi = lax.broadcasted_iota(jnp.int32, (N1, N1), 0)
        perm = (rowi == jnp.broadcast_to(pos_i, (N1, N1))).astype(jnp.bfloat16)

        aq = jnp.dot(q_ref[0].astype(jnp.bfloat16), w0q_ref[...],
                     preferred_element_type=jnp.float32)
        aq_ref[...] = jnp.dot(perm, aq.astype(jnp.bfloat16),
                              preferred_element_type=jnp.float32)

    base = i * TQ
    bd = bd_ref[...]
    for h in range(NSUB):
        start = base + h * SUB

        @pl.when(start < nq)
        def _(h=h, start=start):
            aqr = aq_ref[pl.ds(start, SUB), :]
            enc = jnp.reshape(
                jnp.maximum(aqr[:, None, :] + bd[None, :, :], 0.0),
                (RH, HID)).astype(jnp.bfloat16)
            hb = jnp.dot(enc, w12a_ref[...],
                         preferred_element_type=jnp.float32
                         ).astype(jnp.bfloat16)
            h12 = jnp.maximum(hb + b12a_ref[...].astype(jnp.bfloat16),
                              jnp.bfloat16(0.0))
            h3 = h12.reshape(SUB, N2, 2 * HID)
            s3 = lax.dot_general(wbt_ref[...], h3, (((1,), (2,)), ((), ())),
                                 preferred_element_type=jnp.float32)
            ssc_ref[:, pl.ds(start, SUB), :] = s3 + bout_ref[...][:, :, None]

    @pl.when(i == NSTEP - 1)
    def _():
        qmc = qmc_ref[0]
        cumc = jnp.dot(tril_ref[...], qmc.astype(jnp.bfloat16),
                       preferred_element_type=jnp.float32)
        nqc = jnp.sum(qmc, axis=0, keepdims=True)
        rio = lax.broadcasted_iota(jnp.int32, (N1, 1), 0).astype(jnp.float32)
        posc = jnp.where(qmc > 0.0, cumc - 1.0, nqc + rio - cumc)
        posc_i = posc.astype(jnp.int32)
        coli = lax.broadcasted_iota(jnp.int32, (N1, N1), 1)
        m = ((coli == posc_i) & (qmc > 0.0)).astype(jnp.bfloat16)
        mask2d = qmc * dm_ref[0]
        for o in range(NOUT):
            true_o = jnp.dot(m, ssc_ref[o].astype(jnp.bfloat16),
                             preferred_element_type=jnp.float32)
            out_ref[o, 0] = true_o * mask2d


@jax.jit
def kernel(query, doc, query_mask, doc_mask, W0, b0, Wp1a, bp1a, Wp1b, bp1b,
           Wp2a, bp2a, Wp2b, bp2b):
    w0q = W0[:DIM].astype(jnp.bfloat16)
    w0d = W0[DIM:].astype(jnp.bfloat16)
    w12a = jnp.concatenate([Wp1a, Wp2a], axis=1).astype(jnp.bfloat16)
    wbt = jnp.concatenate([
        jnp.concatenate([Wp1b.T, jnp.zeros((OUT1, HID), Wp1b.dtype)], axis=1),
        jnp.concatenate([jnp.zeros((OUT2, HID), Wp2b.dtype), Wp2b.T], axis=1),
    ], axis=0).astype(jnp.bfloat16)
    b0r = b0.reshape(1, HID)
    b12ar = jnp.concatenate([bp1a, bp2a]).reshape(1, 2 * HID)
    bout = jnp.concatenate([bp1b, bp2b]).reshape(NOUT, 1)
    qmf = query_mask.astype(jnp.float32).reshape(B, 1, N1)
    qmc = query_mask.astype(jnp.float32).reshape(B, N1, 1)
    dm = doc_mask.astype(jnp.float32).reshape(B, 1, N2)
    nq = jnp.sum(query_mask, axis=1).astype(jnp.int32).reshape(B, 1, 1, 1)
    tri = jnp.triu(jnp.ones((N1, N1), jnp.bfloat16))
    tril = jnp.tril(jnp.ones((N1, N1), jnp.bfloat16))

    grid = (B, NSTEP)
    rep = lambda b, i: (0, 0)

    out = pl.pallas_call(
        _pair_kernel,
        grid=grid,
        in_specs=[
            pl.BlockSpec((1, 1, 1, 1), lambda b, i: (b, 0, 0, 0),
                         memory_space=pltpu.SMEM),
            pl.BlockSpec((1, N1, DIM), lambda b, i: (b, 0, 0)),
            pl.BlockSpec((1, N2, DIM), lambda b, i: (b, 0, 0)),
            pl.BlockSpec((1, 1, N1), lambda b, i: (b, 0, 0)),
            pl.BlockSpec((1, N1, 1), lambda b, i: (b, 0, 0)),
            pl.BlockSpec((1, 1, N2), lambda b, i: (b, 0, 0)),
            pl.BlockSpec((N1, N1), rep),
            pl.BlockSpec((N1, N1), rep),
            pl.BlockSpec((DIM, HID), rep),
            pl.BlockSpec((DIM, HID), rep),
            pl.BlockSpec((1, HID), rep),
            pl.BlockSpec((HID, 2 * HID), rep),
            pl.BlockSpec((1, 2 * HID), rep),
            pl.BlockSpec((NOUT, 2 * HID), rep),
            pl.BlockSpec((NOUT, 1), rep),
        ],
        out_specs=pl.BlockSpec((NOUT, 1, N1, N2), lambda b, i: (0, b, 0, 0)),
        out_shape=jax.ShapeDtypeStruct((NOUT, B, N1, N2), jnp.float32),
        scratch_shapes=[pltpu.VMEM((N1, HID), jnp.float32),
                        pltpu.VMEM((N2, HID), jnp.float32),
                        pltpu.VMEM((NOUT, N1, N2), jnp.float32)],
    )(nq, query, doc, qmf, qmc, dm, tri, tril, w0q, w0d, b0r, w12a, b12ar,
      wbt, bout)

    st = out.reshape(NOUT, B * N1 * N2).T
    scores1 = st[:, :OUT1].reshape(B, N1, N2, OUT1)
    scores2 = st[:, OUT1:].reshape(B, N1, N2, OUT2)
    return (scores1, scores2)


# speedup vs baseline: 1.0080x; 1.0080x over previous
import jax
import jax.numpy as jnp
from jax import lax
from jax.experimental import pallas as pl
from jax.experimental.pallas import tpu as pltpu

B, N1, N2, DIM = 4, 256, 256, 128
HID = 128
OUT1, OUT2 = 3, 3
NOUT = OUT1 + OUT2
TQ = 16
SUB = 8
NSUB = TQ // SUB
NSTEP = N1 // TQ
RH = SUB * N2


def _pair_kernel(nq_ref, q_ref, d_ref, qmf_ref, qmc_ref, dm_ref, tri_ref,
                 tril_ref, w0q_ref, w0d_ref, b0_ref, w12a_ref, b12a_ref,
                 wbt_ref, bout_ref, out_ref, aq_ref, bd_ref, ssc_ref):
    i = pl.program_id(1)
    nq = nq_ref[0, 0, 0, 0]

    @pl.when(i == 0)
    def _():
        ssc_ref[...] = jnp.zeros((NOUT, N1, N2), jnp.float32)

        bd_ref[...] = (
            jnp.dot(d_ref[0].astype(jnp.bfloat16), w0d_ref[...],
                    preferred_element_type=jnp.float32) + b0_ref[...]
        )

        qmv = qmf_ref[0]
        cum = jnp.dot(qmv.astype(jnp.bfloat16), tri_ref[...],
                      preferred_element_type=jnp.float32)
        nqv = jnp.sum(qmv, axis=1, keepdims=True)
        lane = lax.broadcasted_iota(jnp.int32, (1, N1), 1).astype(jnp.float32)
        pos = jnp.where(qmv > 0.0, cum - 1.0, nqv + lane - cum)
        pos_i = pos.astype(jnp.int32)

        rowi = lax.broadcasted_iota(jnp.int32, (N1, N1), 0)
        perm = (rowi == jnp.broadcast_to(pos_i, (N1, N1))).astype(jnp.bfloat16)

        aq = jnp.dot(q_ref[0].astype(jnp.bfloat16), w0q_ref[...],
                     preferred_element_type=jnp.float32)
        aq_ref[...] = jnp.dot(perm, aq.astype(jnp.bfloat16),
                              preferred_element_type=jnp.float32)

    base = i * TQ
    bd = bd_ref[...]
    for h in range(NSUB):
        start = base + h * SUB

        @pl.when(start < nq)
        def _(h=h, start=start):
            aqr = aq_ref[pl.ds(start, SUB), :]
            enc = jnp.reshape(
                jnp.maximum(aqr[:, None, :] + bd[None, :, :], 0.0),
                (RH, HID)).astype(jnp.bfloat16)
            h12 = jnp.maximum(
                jnp.dot(enc, w12a_ref[...],
                        preferred_element_type=jnp.float32)
                + b12a_ref[...], 0.0).astype(jnp.bfloat16)
            for t in range(SUB):
                h12t = h12[t * N2:(t + 1) * N2, :]
                stt = lax.dot_general(wbt_ref[...], h12t,
                                      (((1,), (1,)), ((), ())),
                                      preferred_element_type=jnp.float32)
                stt = stt + bout_ref[...]
                ssc_ref[:, pl.ds(start + t, 1), :] = stt.reshape(NOUT, 1, N2)

    @pl.when(i == NSTEP - 1)
    def _():
        qmc = qmc_ref[0]
        cumc = jnp.dot(tril_ref[...], qmc.astype(jnp.bfloat16),
                       preferred_element_type=jnp.float32)
        nqc = jnp.sum(qmc, axis=0, keepdims=True)
        rio = lax.broadcasted_iota(jnp.int32, (N1, 1), 0).astype(jnp.float32)
        posc = jnp.where(qmc > 0.0, cumc - 1.0, nqc + rio - cumc)
        posc_i = posc.astype(jnp.int32)
        coli = lax.broadcasted_iota(jnp.int32, (N1, N1), 1)
        m = ((coli == posc_i) & (qmc > 0.0)).astype(jnp.bfloat16)
        mask2d = qmc * dm_ref[0]
        for o in range(NOUT):
            true_o = jnp.dot(m, ssc_ref[o].astype(jnp.bfloat16),
                             preferred_element_type=jnp.float32)
            out_ref[o, 0] = true_o * mask2d


@jax.jit
def kernel(query, doc, query_mask, doc_mask, W0, b0, Wp1a, bp1a, Wp1b, bp1b,
           Wp2a, bp2a, Wp2b, bp2b):
    w0q = W0[:DIM].astype(jnp.bfloat16)
    w0d = W0[DIM:].astype(jnp.bfloat16)
    w12a = jnp.concatenate([Wp1a, Wp2a], axis=1).astype(jnp.bfloat16)
    wbt = jnp.concatenate([
        jnp.concatenate([Wp1b.T, jnp.zeros((OUT1, HID), Wp1b.dtype)], axis=1),
        jnp.concatenate([jnp.zeros((OUT2, HID), Wp2b.dtype), Wp2b.T], axis=1),
    ], axis=0).astype(jnp.bfloat16)
    b0r = b0.reshape(1, HID)
    b12ar = jnp.concatenate([bp1a, bp2a]).reshape(1, 2 * HID)
    bout = jnp.concatenate([bp1b, bp2b]).reshape(NOUT, 1)
    qmf = query_mask.astype(jnp.float32).reshape(B, 1, N1)
    qmc = query_mask.astype(jnp.float32).reshape(B, N1, 1)
    dm = doc_mask.astype(jnp.float32).reshape(B, 1, N2)
    nq = jnp.sum(query_mask, axis=1).astype(jnp.int32).reshape(B, 1, 1, 1)
    tri = jnp.triu(jnp.ones((N1, N1), jnp.bfloat16))
    tril = jnp.tril(jnp.ones((N1, N1), jnp.bfloat16))

    grid = (B, NSTEP)
    rep = lambda b, i: (0, 0)

    out = pl.pallas_call(
        _pair_kernel,
        grid=grid,
        in_specs=[
            pl.BlockSpec((1, 1, 1, 1), lambda b, i: (b, 0, 0, 0),
                         memory_space=pltpu.SMEM),
            pl.BlockSpec((1, N1, DIM), lambda b, i: (b, 0, 0)),
            pl.BlockSpec((1, N2, DIM), lambda b, i: (b, 0, 0)),
            pl.BlockSpec((1, 1, N1), lambda b, i: (b, 0, 0)),
            pl.BlockSpec((1, N1, 1), lambda b, i: (b, 0, 0)),
            pl.BlockSpec((1, 1, N2), lambda b, i: (b, 0, 0)),
            pl.BlockSpec((N1, N1), rep),
            pl.BlockSpec((N1, N1), rep),
            pl.BlockSpec((DIM, HID), rep),
            pl.BlockSpec((DIM, HID), rep),
            pl.BlockSpec((1, HID), rep),
            pl.BlockSpec((HID, 2 * HID), rep),
            pl.BlockSpec((1, 2 * HID), rep),
            pl.BlockSpec((NOUT, 2 * HID), rep),
            pl.BlockSpec((NOUT, 1), rep),
        ],
        out_specs=pl.BlockSpec((NOUT, 1, N1, N2), lambda b, i: (0, b, 0, 0)),
        out_shape=jax.ShapeDtypeStruct((NOUT, B, N1, N2), jnp.float32),
        scratch_shapes=[pltpu.VMEM((N1, HID), jnp.float32),
                        pltpu.VMEM((N2, HID), jnp.float32),
                        pltpu.VMEM((NOUT, N1, N2), jnp.float32)],
    )(nq, query, doc, qmf, qmc, dm, tri, tril, w0q, w0d, b0r, w12a, b12ar,
      wbt, bout)

    st = out.reshape(NOUT, B * N1 * N2).T
    scores1 = st[:, :OUT1].reshape(B, N1, N2, OUT1)
    scores2 = st[:, OUT1:].reshape(B, N1, N2, OUT2)
    return (scores1, scores2)
